# baseline (speedup 1.0000x reference)
import functools

import jax
import jax.numpy as jnp
from jax import lax
from jax.experimental import pallas as pl
from jax.experimental.pallas import tpu as pltpu, tpu_sc as plsc

NUM_EMB = 1000000
DIM = 64
BATCH = 16384
L = 16

_info = plsc.get_sparse_core_info()
_NC, _NS = _info.num_cores, _info.num_subcores
_NW = _NC * _NS
_BPW = BATCH // _NW

_COLS_PW = 31232
_CHW = 384
_NCH = 83
_TRASH = BATCH
_SENT = 2 ** 30


def _phase1(idx_hbm, perm_hbm, tableT_hbm, inter_hbm,
            idx_v, perm_v, buf_a, buf_b, stg_a, stg_b, ids_a, ids_b,
            sem_a, sem_b, sem_sc):
    w = lax.axis_index("s") * _NC + lax.axis_index("c")
    range_start = w * _COLS_PW
    range_end = jnp.where(w == _NW - 1, NUM_EMB, range_start + _COLS_PW)
    ones = jnp.full((L,), 1, jnp.int32)
    iota16 = lax.iota(jnp.int32, L)

    pltpu.sync_copy(idx_hbm, idx_v.at[pl.ds(0, BATCH)])
    pltpu.sync_copy(perm_hbm, perm_v.at[pl.ds(0, BATCH)])
    for j in range(8):
        idx_v[pl.ds(BATCH + j * L, L)] = jnp.full((L,), _SENT, jnp.int32)
        perm_v[pl.ds(BATCH + j * L, L)] = jnp.full((L,), _TRASH, jnp.int32)

    rs_vec = ones * range_start

    def cnt(i, acc):
        m = idx_v[pl.ds(i * L, L)] < rs_vec
        return acc + plsc.all_reduce_population_count(m)

    p0 = lax.fori_loop(0, BATCH // L, cnt, jnp.zeros((L,), jnp.int32))[0]

    def fire(c, buf, sem):
        col0 = range_start + c * _CHW
        pltpu.async_copy(tableT_hbm.at[:, pl.ds(col0, _CHW)], buf, sem)

    def wait(buf, sem):
        pltpu.make_async_copy(
            tableT_hbm.at[:, pl.ds(0, _CHW)], buf, sem).wait()

    def drain_one(stg, ids_ref):
        pltpu.make_async_copy(stg.at[pl.ds(0, L)],
                              inter_hbm.at[ids_ref.at[0]], sem_sc).wait()

    def process(c, buf, stg, ids_ref, p, g_prev):
        def drain(g, carry):
            drain_one(stg, ids_ref)
            return carry

        lax.fori_loop(0, g_prev, drain, jnp.int32(0))

        col0 = range_start + c * _CHW
        chunk_end = jnp.minimum(col0 + _CHW, range_end)
        ce_vec = ones * chunk_end
        c0_vec = ones * col0

        def cntw(j, acc):
            m = idx_v[pl.ds(p + j * L, L)] < ce_vec
            return acc + plsc.all_reduce_population_count(m)

        n = lax.fori_loop(0, 8, cntw, jnp.zeros((L,), jnp.int32))[0]
        ng = (n + L - 1) // L

        def grp(g, carry):
            base = p + g * L
            wv = idx_v[pl.ds(base, L)]
            pv = perm_v[pl.ds(base, L)]
            m = wv < ce_vec
            cols = jnp.where(m, wv - c0_vec, 0)
            ids = jnp.where(m, pv, ones * _TRASH)
            ids_ref[g] = ids
            rows16 = g * L + iota16
            for r in range(DIM):
                vals = plsc.load_gather(
                    buf, [jnp.full((L,), r, jnp.int32), cols])
                plsc.store_scatter(
                    stg, [rows16, jnp.full((L,), r, jnp.int32)], vals)
            pltpu.async_copy(stg.at[pl.ds(g * L, L)],
                             inter_hbm.at[ids_ref.at[g]], sem_sc)
            return carry

        lax.fori_loop(0, ng, grp, jnp.int32(0))
        return p + n, ng

    fire(0, buf_a, sem_a)
    fire(1, buf_b, sem_b)

    def pair(t, carry):
        p, ga, gb = carry
        c0 = 2 * t
        wait(buf_a, sem_a)
        p, ga = process(c0, buf_a, stg_a, ids_a, p, ga)
        fire(c0 + 2, buf_a, sem_a)
        wait(buf_b, sem_b)
        p, gb = process(c0 + 1, buf_b, stg_b, ids_b, p, gb)
        fire(c0 + 3, buf_b, sem_b)
        return p, ga, gb

    p, ga, gb = lax.fori_loop(
        0, 40, pair, (p0, jnp.int32(0), jnp.int32(0)))

    wait(buf_a, sem_a)
    p, ga = process(80, buf_a, stg_a, ids_a, p, ga)
    fire(82, buf_a, sem_a)
    wait(buf_b, sem_b)
    p, gb = process(81, buf_b, stg_b, ids_b, p, gb)
    wait(buf_a, sem_a)
    p, ga = process(82, buf_a, stg_a, ids_a, p, ga)

    def drain_a(g, carry):
        drain_one(stg_a, ids_a)
        return carry

    def drain_b(g, carry):
        drain_one(stg_b, ids_b)
        return carry

    lax.fori_loop(0, ga, drain_a, jnp.int32(0))
    lax.fori_loop(0, gb, drain_b, jnp.int32(0))


def _phase2(inter_hbm, out_hbm, slab_v, out_v):
    w = lax.axis_index("s") * _NC + lax.axis_index("c")
    base = w * _BPW
    pltpu.sync_copy(inter_hbm.at[pl.ds(base, _BPW), :], slab_v)

    def compact(i, carry):
        for kk in range(DIM // L):
            out_v[i, pl.ds(kk * L, L)] = slab_v[i, pl.ds(kk * L, L)]
        return carry

    lax.fori_loop(0, _BPW, compact, jnp.int32(0))
    pltpu.sync_copy(out_v, out_hbm.at[pl.ds(base, _BPW)])


@jax.jit
def _gather(idx_s, order, weight):
    mesh = plsc.VectorSubcoreMesh(core_axis_name="c", subcore_axis_name="s")
    p1 = functools.partial(
        pl.kernel,
        mesh=mesh,
        out_type=jax.ShapeDtypeStruct((BATCH + L, 128), jnp.float32),
        scratch_types=[
            pltpu.VMEM((BATCH + 8 * L,), jnp.int32),
            pltpu.VMEM((BATCH + 8 * L,), jnp.int32),
            pltpu.VMEM((DIM, _CHW), jnp.float32),
            pltpu.VMEM((DIM, _CHW), jnp.float32),
            pltpu.VMEM((8 * L, 128), jnp.float32),
            pltpu.VMEM((8 * L, 128), jnp.float32),
            pltpu.VMEM((8, L), jnp.int32),
            pltpu.VMEM((8, L), jnp.int32),
            pltpu.SemaphoreType.DMA,
            pltpu.SemaphoreType.DMA,
            pltpu.SemaphoreType.DMA,
        ],
        compiler_params=pltpu.CompilerParams(needs_layout_passes=False),
    )(_phase1)
    inter = p1(idx_s, order, weight.T)

    p2 = functools.partial(
        pl.kernel,
        mesh=mesh,
        out_type=jax.ShapeDtypeStruct((BATCH, DIM), jnp.float32),
        scratch_types=[pltpu.VMEM((_BPW, 128), jnp.float32),
                       pltpu.VMEM((_BPW, DIM), jnp.float32)],
    )(_phase2)
    return p2(inter)


def kernel(x, weight):
    idx = jnp.remainder(x, NUM_EMB).astype(jnp.int32)
    idx_s = jnp.sort(idx)
    order = jnp.argsort(idx).astype(jnp.int32)
    return _gather(idx_s, order, weight)
